# baseline (speedup 1.0000x reference)
import functools

import jax
import jax.numpy as jnp
from jax import lax
from jax.experimental import pallas as pl
from jax.experimental.pallas import tpu as pltpu
from jax.experimental.pallas import tpu_sc as plsc

_NC = 2
_NS = 16
_NW = _NC * _NS
_CH = 128


def _sc_gather(ut2, it2, uidx, iidx):
  n_ch = uidx.shape[1]
  b_per_w = n_ch * _CH
  B = _NW * b_per_w

  mesh = plsc.VectorSubcoreMesh(core_axis_name="c", subcore_axis_name="s")

  @functools.partial(
      pl.kernel,
      out_type=[
          jax.ShapeDtypeStruct((B, 128), jnp.float32),
          jax.ShapeDtypeStruct((B, 128), jnp.float32),
      ],
      mesh=mesh,
      scratch_types=[
          pltpu.VMEM((n_ch, _CH), jnp.int32),
          pltpu.VMEM((n_ch, _CH), jnp.int32),
          pltpu.VMEM((2, _CH, 128), jnp.float32),
          pltpu.VMEM((2, _CH, 128), jnp.float32),
      ] + [pltpu.SemaphoreType.DMA] * 8,
  )
  def k(ut, it, ui, ii, uo, io, ui_v, ii_v, ub, ib,
        gu0, gu1, gi0, gi1, ou0, ou1, oi0, oi1):
    gu = (gu0, gu1)
    gi = (gi0, gi1)
    ou = (ou0, ou1)
    oi = (oi0, oi1)
    c = lax.axis_index("c")
    s = lax.axis_index("s")
    wid = s * _NC + c
    base = wid * b_per_w
    pltpu.sync_copy(ui.at[wid], ui_v)
    pltpu.sync_copy(ii.at[wid], ii_v)

    g_cps = [None, None]
    o_cps = [None, None]

    def start_gathers(j):
      b = j % 2
      g_cps[b] = (
          pltpu.async_copy(ut.at[ui_v.at[j]], ub.at[b], gu[b]),
          pltpu.async_copy(it.at[ii_v.at[j]], ib.at[b], gi[b]),
      )

    def drain_and_store(j):
      b = j % 2
      g_cps[b][0].wait()
      g_cps[b][1].wait()
      o_cps[b] = (
          pltpu.async_copy(ub.at[b], uo.at[pl.ds(base + j * _CH, _CH)], ou[b]),
          pltpu.async_copy(ib.at[b], io.at[pl.ds(base + j * _CH, _CH)], oi[b]),
      )

    start_gathers(0)
    if n_ch > 1:
      start_gathers(1)
    for j in range(n_ch):
      drain_and_store(j)
      if j + 2 < n_ch:
        b = j % 2
        o_cps[b][0].wait()
        o_cps[b][1].wait()
        o_cps[b] = None
        start_gathers(j + 2)
    for b in range(2):
      if o_cps[b] is not None:
        o_cps[b][0].wait()
        o_cps[b][1].wait()

  return k(ut2, it2, uidx, iidx)


def _leaky(x):
  return jnp.where(x >= 0, x, 0.01 * x)


def _tc_mlp(xu_raw, xi_raw, sub_u, sub_i, feats, pg_idx, cg_idx, in_idx,
            pg_table, cg_table, in_table, wu_rep, wi_rep, w_tail,
            Ws, bs, *, interpret=False):
  B = xu_raw.shape[0]
  BM = 1024
  grid = (B // BM,)
  n_pg = pg_table.shape[0]
  n_cg = cg_table.shape[0]
  n_in = in_table.shape[0]

  def body(xu_ref, xi_ref, su_ref, si_ref, f_ref, pg_ref, cg_ref, in_ref,
           pgt_ref, cgt_ref, int_ref, wu_ref, wi_ref, wt_ref, *wb_refs):
    o_ref = wb_refs[-1]
    w_refs = wb_refs[0:5]
    b_refs = wb_refs[5:11]
    lane = lax.broadcasted_iota(jnp.int32, (1, 128), 1)
    mu = (lane // 16 == su_ref[...]).astype(jnp.float32)
    mi = (lane // 32 == si_ref[...]).astype(jnp.float32)
    xu = xu_ref[...] * mu
    xi = xi_ref[...] * mi
    oh_pg = (pg_ref[...] == lax.broadcasted_iota(jnp.int32, (1, n_pg), 1)
             ).astype(jnp.float32)
    oh_cg = (cg_ref[...] == lax.broadcasted_iota(jnp.int32, (1, n_cg), 1)
             ).astype(jnp.float32)
    oh_in = (in_ref[...] == lax.broadcasted_iota(jnp.int32, (1, n_in), 1)
             ).astype(jnp.float32)
    pgE = jnp.dot(oh_pg, pgt_ref[...], preferred_element_type=jnp.float32)
    cgE = jnp.dot(oh_cg, cgt_ref[...], preferred_element_type=jnp.float32)
    inE = jnp.dot(oh_in, int_ref[...], preferred_element_type=jnp.float32)
    xs = jnp.concatenate([pgE, cgE, inE, f_ref[...]], axis=1)
    x = (jnp.dot(xu, wu_ref[...], preferred_element_type=jnp.float32)
         + jnp.dot(xi, wi_ref[...], preferred_element_type=jnp.float32)
         + jnp.dot(xs, wt_ref[...], preferred_element_type=jnp.float32)
         + b_refs[0][...])
    x = _leaky(x)
    for wr, br in zip(w_refs, b_refs[1:]):
      x = jnp.dot(x, wr[...], preferred_element_type=jnp.float32) + br[...]
      x = _leaky(x)
    o_ref[...] = x

  def row_spec(d):
    return pl.BlockSpec((BM, d), lambda i: (i, 0))

  def full_spec(shape):
    nd = len(shape)
    if nd == 1:
      return pl.BlockSpec(shape, lambda i: (0,))
    return pl.BlockSpec(shape, lambda i: (0, 0))

  in_specs = [
      row_spec(128), row_spec(128),
      row_spec(1), row_spec(1),
      row_spec(feats.shape[1]),
      row_spec(1), row_spec(1), row_spec(1),
      full_spec(pg_table.shape), full_spec(cg_table.shape),
      full_spec(in_table.shape),
      full_spec(wu_rep.shape), full_spec(wi_rep.shape), full_spec(w_tail.shape),
  ]
  for W in Ws[1:]:
    in_specs.append(full_spec(W.shape))
  for b in bs:
    in_specs.append(full_spec(b.shape))

  out_dim = Ws[-1].shape[1]
  return pl.pallas_call(
      body,
      grid=grid,
      in_specs=in_specs,
      out_specs=pl.BlockSpec((BM, out_dim), lambda i: (i, 0)),
      out_shape=jax.ShapeDtypeStruct((B, out_dim), jnp.float32),
      interpret=interpret,
  )(xu_raw, xi_raw, sub_u, sub_i, feats, pg_idx, cg_idx, in_idx,
    pg_table, cg_table, in_table, wu_rep, wi_rep, w_tail, *Ws[1:], *bs)


def kernel(user_input, item_input, prices, sales_channels, club_status,
           age_groups, product_groups, color_groups, index_name,
           user_table, item_table, pg_table, cg_table, in_table, Ws, bs):
  B = user_input.shape[0]
  n_ch = B // (_NW * _CH)
  ui32 = user_input.astype(jnp.int32)
  ii32 = item_input.astype(jnp.int32)
  uidx = (ui32 >> 3).reshape(_NW, n_ch, _CH)
  iidx = (ii32 >> 2).reshape(_NW, n_ch, _CH)
  sub_u = (ui32 & 7).reshape(B, 1)
  sub_i = (ii32 & 3).reshape(B, 1)

  ut2 = user_table.reshape(-1, 128)
  it2 = item_table.reshape(-1, 128)
  xu_raw, xi_raw = _sc_gather(ut2, it2, uidx, iidx)

  W1 = Ws[0]
  wu_rep = jnp.concatenate([W1[0:16]] * 8, axis=0)
  wi_rep = jnp.concatenate([W1[16:48]] * 4, axis=0)
  w_tail = W1[48:76]

  feats = jnp.stack([prices, sales_channels, club_status, age_groups], axis=1)
  pg = product_groups.astype(jnp.int32).reshape(B, 1)
  cg = color_groups.astype(jnp.int32).reshape(B, 1)
  inm = index_name.astype(jnp.int32).reshape(B, 1)
  return _tc_mlp(xu_raw, xi_raw, sub_u, sub_i, feats, pg, cg, inm,
                 pg_table, cg_table, in_table, wu_rep, wi_rep, w_tail,
                 Ws, bs)
